# baseline (speedup 1.0000x reference)
import functools

import jax
import jax.numpy as jnp
import numpy as np
from jax import lax
from jax.experimental import pallas as pl
from jax.experimental.pallas import tpu as pltpu
from jax.experimental.pallas import tpu_sc as plsc

_N = 10000
_E = 320000
_DM = 768
_NP = 10240
_RPAD = 2560
_RPT = _RPAD // 32
_RPT_CS = _RPAD // 16
_ROWS_PT = _NP // 16
_BLK = 512
_G = _NP // _BLK

_mesh = plsc.VectorSubcoreMesh(core_axis_name="c", subcore_axis_name="s")
_sc_params = pltpu.CompilerParams(use_tc_tiling_on_sc=False)


def _sc_agg(table2, src2d, dst2d, dh):
    @functools.partial(
        pl.kernel,
        out_type=jax.ShapeDtypeStruct((2, _NP, dh), jnp.float32),
        mesh=_mesh,
        scratch_types=[
            pltpu.VMEM((40, 128), jnp.int32),
            pltpu.VMEM((40, 128), jnp.int32),
            pltpu.VMEM((8, 128, dh), jnp.float32),
            pltpu.VMEM((128, dh), jnp.float32),
            pltpu.VMEM_SHARED((_NP, dh), jnp.float32),
        ] + [pltpu.SemaphoreType.DMA] * 16,
        compiler_params=_sc_params,
    )
    def k(table_hbm, src_hbm, dst_hbm, out_hbm, sidx, didx, rows4, zbuf, acc,
          *sems):
        c = lax.axis_index("c")
        s = lax.axis_index("s")
        gsems = sems[:8]
        ssems = sems[8:]

        @pl.loop(0, 128)
        def _(r):
            for kk in range(dh // 16):
                zbuf[r, pl.ds(kk * 16, 16)] = jnp.zeros((16,), jnp.float32)

        @pl.loop(0, 5)
        def _(t):
            pltpu.sync_copy(zbuf, acc.at[pl.ds(s * _ROWS_PT + t * 128, 128)])

        plsc.subcore_barrier()

        tbl = table_hbm.at[c]

        @pl.loop(0, 4)
        def _(ph):
            base = s * _RPT_CS + ph * 40
            pltpu.sync_copy(src_hbm.at[pl.ds(base, 40)], sidx)
            pltpu.sync_copy(dst_hbm.at[pl.ds(base, 40)], didx)
            for b in range(4):
                pltpu.async_copy(tbl.at[sidx.at[b]], rows4.at[b], gsems[b])

            @pl.loop(0, 40, step=8)
            def _(j):
                for b in range(8):
                    g = j + b
                    bh = (b + 4) % 8
                    pltpu.make_async_copy(tbl.at[sidx.at[g]], rows4.at[b],
                                          gsems[b]).wait()
                    pltpu.async_copy(rows4.at[b], acc.at[didx.at[g]],
                                     ssems[b], add=True)

                    @pl.when(g + 4 < 40)
                    def _():
                        @pl.when(g >= 4)
                        def _():
                            pltpu.make_async_copy(
                                rows4.at[bh], acc.at[didx.at[g - 4]],
                                ssems[bh]).wait()
                        pltpu.async_copy(tbl.at[sidx.at[g + 4]], rows4.at[bh],
                                         gsems[bh])

            for b in range(8):
                pltpu.make_async_copy(rows4.at[b], acc.at[didx.at[32 + b]],
                                      ssems[b]).wait()

        plsc.subcore_barrier()

        @pl.loop(0, 5)
        def _(t):
            sl = pl.ds(s * _ROWS_PT + t * 128, 128)
            pltpu.sync_copy(acc.at[sl], zbuf)
            pltpu.sync_copy(zbuf, out_hbm.at[c, sl])

    return k(table2, src2d, dst2d)


def _sc_deg(dst2d):
    @functools.partial(
        pl.kernel,
        out_type=jax.ShapeDtypeStruct((2, _NP), jnp.float32),
        mesh=_mesh,
        scratch_types=[
            pltpu.VMEM((_RPT, 128), jnp.int32),
            pltpu.VMEM((128,), jnp.float32),
            pltpu.VMEM((128,), jnp.float32),
            pltpu.VMEM_SHARED((_NP,), jnp.float32),
        ],
        compiler_params=_sc_params,
    )
    def k(dst_hbm, out_hbm, didx, ones, buf, acc):
        c = lax.axis_index("c")
        s = lax.axis_index("s")
        wid = c * 16 + s

        for kk in range(8):
            ones[pl.ds(kk * 16, 16)] = jnp.ones((16,), jnp.float32)
            buf[pl.ds(kk * 16, 16)] = jnp.zeros((16,), jnp.float32)

        @pl.loop(0, 5)
        def _(t):
            pltpu.sync_copy(buf, acc.at[pl.ds(s * _ROWS_PT + t * 128, 128)])

        plsc.subcore_barrier()

        pltpu.sync_copy(dst_hbm.at[pl.ds(wid * _RPT, _RPT)], didx)

        @pl.loop(0, _RPT)
        def _(j):
            pltpu.sync_copy(ones, acc.at[didx.at[j]], add=True)

        plsc.subcore_barrier()

        @pl.loop(0, 5)
        def _(t):
            sl = pl.ds(s * _ROWS_PT + t * 128, 128)
            pltpu.sync_copy(acc.at[sl], buf)
            pltpu.sync_copy(buf, out_hbm.at[c, sl])

    return k(dst2d)


def _ln(h, g, b):
    mu = jnp.mean(h, axis=1, keepdims=True)
    dd = h - mu
    var = jnp.mean(dd * dd, axis=1, keepdims=True)
    return dd * lax.rsqrt(var + 1e-5) * g + b


def _gelu_new(h):
    c = np.sqrt(2.0 / np.pi).astype(np.float32)
    return 0.5 * h * (1.0 + jnp.tanh(c * (h + 0.044715 * h * h * h)))


def _bf(a):
    return a.astype(jnp.bfloat16)


def _dot(a, w):
    return jnp.dot(_bf(a), w, preferred_element_type=jnp.float32)


def _tc_big_body(pr, xr, disr, w1, b1, g1, e1, wv, bv, wo, bo, g2, e2,
                 wfc, bfc, wp, bp, w2, o_ref):
    dis = disr[...]
    u = dis * pr[...] + (dis * dis) * xr[...]
    h = jax.nn.relu(_dot(u, w1[...]) + b1[...])
    for l in range(2):
        a = _ln(h, g1[l], e1[l])
        v = _dot(a, wv[l]) + bv[l]
        h = h + _dot(v, wo[l]) + bo[l]
        m = _ln(h, g2[l], e2[l])
        f = _gelu_new(_dot(m, wfc[l]) + bfc[l])
        h = h + _dot(f, wp[l]) + bp[l]
    r = jax.nn.relu(h)
    o_ref[...] = _dot(r, w2[...]) * dis


def _tc_fin_body(qr, hwdr, disr, b2r, o_ref):
    dis = disr[...]
    z = dis * (qr[...] + hwdr[...]) + b2r[...]
    col = lax.broadcasted_iota(jnp.int32, z.shape, 1)
    zm = jnp.where(col < 40, z, -1e30)
    mx = jnp.max(zm, axis=1, keepdims=True)
    lse = jnp.log(jnp.sum(jnp.exp(zm - mx), axis=1, keepdims=True)) + mx
    o_ref[...] = z - lse


def _row_spec(d):
    return pl.BlockSpec((_BLK, d), lambda i: (i, 0))


def _full(shape):
    nd = len(shape)
    return pl.BlockSpec(shape, lambda i, _nd=nd: (0,) * _nd)


def _part_spec(j, d):
    return pl.BlockSpec((1, _BLK, d), lambda i, _j=j: (_j, i, 0))


def kernel(x, edge_index, W1, b1, ln1_g, ln1_b, Wqkv, bqkv, Wo, bo,
           ln2_g, ln2_b, Wfc, bfc, Wp, bp, W2, b2):
    f32 = jnp.float32
    src, dst = edge_index[0], edge_index[1]
    pad_e = _RPAD * 128 - _E
    padv = jnp.full((pad_e,), _N, jnp.int32)
    src2d = jnp.concatenate([src, padv]).reshape(_RPAD, 128)
    dst2d = jnp.concatenate([dst, padv]).reshape(_RPAD, 128)
    xp = jnp.pad(x, ((0, _NP - _N), (0, 0)))

    degp = _sc_deg(dst2d)
    dis = lax.rsqrt(degp[0] + degp[1] + 1.0)
    dis2 = dis[:, None]
    hs0 = xp * dis2

    hs0s = hs0.reshape(_NP, 2, 64).transpose(1, 0, 2)
    ap = _sc_agg(hs0s, src2d, dst2d, 64)
    agg = ap.transpose(1, 0, 2).reshape(_NP, 128)

    Wv = _bf(Wqkv[:, :, 2 * _DM:])
    bv = bqkv[:, 2 * _DM:]
    W2p = _bf(jnp.pad(W2, ((0, 0), (0, 24))))
    b2p = jnp.pad(b2, (0, 24))[None]

    hwd = pl.pallas_call(
        _tc_big_body,
        grid=(_G,),
        in_specs=[
            _row_spec(128),
            _row_spec(128), _row_spec(1),
            _full((128, _DM)), _full((1, _DM)),
            _full((2, _DM)), _full((2, _DM)),
            _full((2, _DM, _DM)), _full((2, _DM)),
            _full((2, _DM, _DM)), _full((2, _DM)),
            _full((2, _DM)), _full((2, _DM)),
            _full((2, _DM, 3072)), _full((2, 3072)),
            _full((2, 3072, _DM)), _full((2, _DM)),
            _full((_DM, 64)),
        ],
        out_specs=_row_spec(64),
        out_shape=jax.ShapeDtypeStruct((_NP, 64), f32),
    )(agg, xp, dis2, _bf(W1), b1[None], ln1_g, ln1_b, Wv, bv,
      _bf(Wo), bo, ln2_g, ln2_b, _bf(Wfc), bfc, _bf(Wp), bp, W2p)

    hwds = hwd.reshape(_NP, 2, 32).transpose(1, 0, 2)
    qp = _sc_agg(hwds, src2d, dst2d, 32)
    q = qp.transpose(1, 0, 2).reshape(_NP, 64)

    out64 = pl.pallas_call(
        _tc_fin_body,
        grid=(_G,),
        in_specs=[
            _row_spec(64),
            _row_spec(64), _row_spec(1), _full((1, 64)),
        ],
        out_specs=_row_spec(64),
        out_shape=jax.ShapeDtypeStruct((_NP, 64), f32),
    )(q, hwd, dis2, b2p)

    return out64[:_N, :40]


# speedup vs baseline: 1.0157x; 1.0157x over previous
import functools

import jax
import jax.numpy as jnp
import numpy as np
from jax import lax
from jax.experimental import pallas as pl
from jax.experimental.pallas import tpu as pltpu
from jax.experimental.pallas import tpu_sc as plsc

_N = 10000
_E = 320000
_DM = 768
_NP = 10240
_RPAD = 2560
_RPT = _RPAD // 32
_RPT_CS = _RPAD // 16
_ROWS_PT = _NP // 16
_BLK = 1024
_G = _NP // _BLK

_mesh = plsc.VectorSubcoreMesh(core_axis_name="c", subcore_axis_name="s")
_sc_params = pltpu.CompilerParams(use_tc_tiling_on_sc=False)


def _sc_agg(table2, src2d, dst2d, dh):
    @functools.partial(
        pl.kernel,
        out_type=jax.ShapeDtypeStruct((2, _NP, dh), jnp.float32),
        mesh=_mesh,
        scratch_types=[
            pltpu.VMEM((_RPT_CS, 128), jnp.int32),
            pltpu.VMEM((_RPT_CS, 128), jnp.int32),
            pltpu.VMEM((4, 128, dh), jnp.float32),
            pltpu.VMEM((128, dh), jnp.float32),
            pltpu.VMEM_SHARED((_NP, dh), jnp.float32),
        ] + [pltpu.SemaphoreType.DMA] * 4,
        compiler_params=_sc_params,
    )
    def k(table_hbm, src_hbm, dst_hbm, out_hbm, sidx, didx, rows4, zbuf, acc,
          *sems):
        c = lax.axis_index("c")
        s = lax.axis_index("s")
        gsems = sems

        @pl.loop(0, 128)
        def _(r):
            for kk in range(dh // 16):
                zbuf[r, pl.ds(kk * 16, 16)] = jnp.zeros((16,), jnp.float32)

        @pl.loop(0, 5)
        def _(t):
            pltpu.sync_copy(zbuf, acc.at[pl.ds(s * _ROWS_PT + t * 128, 128)])

        plsc.subcore_barrier()

        pltpu.sync_copy(src_hbm.at[pl.ds(s * _RPT_CS, _RPT_CS)], sidx)
        pltpu.sync_copy(dst_hbm.at[pl.ds(s * _RPT_CS, _RPT_CS)], didx)

        tbl = table_hbm.at[c]
        for b in range(4):
            pltpu.async_copy(tbl.at[sidx.at[b]], rows4.at[b], gsems[b])

        @pl.loop(0, _RPT_CS, step=4)
        def _(j):
            for b in range(4):
                g = j + b
                pltpu.make_async_copy(tbl.at[sidx.at[g]], rows4.at[b],
                                      gsems[b]).wait()
                pltpu.sync_copy(rows4.at[b], acc.at[didx.at[g]], add=True)

                @pl.when(g + 4 < _RPT_CS)
                def _():
                    pltpu.async_copy(tbl.at[sidx.at[g + 4]], rows4.at[b],
                                     gsems[b])

        plsc.subcore_barrier()

        @pl.loop(0, 5)
        def _(t):
            sl = pl.ds(s * _ROWS_PT + t * 128, 128)
            pltpu.sync_copy(acc.at[sl], zbuf)
            pltpu.sync_copy(zbuf, out_hbm.at[c, sl])

    return k(table2, src2d, dst2d)


def _sc_deg(dst2d):
    @functools.partial(
        pl.kernel,
        out_type=jax.ShapeDtypeStruct((2, _NP), jnp.float32),
        mesh=_mesh,
        scratch_types=[
            pltpu.VMEM((_RPT, 128), jnp.int32),
            pltpu.VMEM((128,), jnp.float32),
            pltpu.VMEM((128,), jnp.float32),
            pltpu.VMEM_SHARED((_NP,), jnp.float32),
        ],
        compiler_params=_sc_params,
    )
    def k(dst_hbm, out_hbm, didx, ones, buf, acc):
        c = lax.axis_index("c")
        s = lax.axis_index("s")
        wid = c * 16 + s

        for kk in range(8):
            ones[pl.ds(kk * 16, 16)] = jnp.ones((16,), jnp.float32)
            buf[pl.ds(kk * 16, 16)] = jnp.zeros((16,), jnp.float32)

        @pl.loop(0, 5)
        def _(t):
            pltpu.sync_copy(buf, acc.at[pl.ds(s * _ROWS_PT + t * 128, 128)])

        plsc.subcore_barrier()

        pltpu.sync_copy(dst_hbm.at[pl.ds(wid * _RPT, _RPT)], didx)

        @pl.loop(0, _RPT)
        def _(j):
            pltpu.sync_copy(ones, acc.at[didx.at[j]], add=True)

        plsc.subcore_barrier()

        @pl.loop(0, 5)
        def _(t):
            sl = pl.ds(s * _ROWS_PT + t * 128, 128)
            pltpu.sync_copy(acc.at[sl], buf)
            pltpu.sync_copy(buf, out_hbm.at[c, sl])

    return k(dst2d)


def _bf(a):
    return a.astype(jnp.bfloat16)


def _dot(a, w):
    return jnp.dot(_bf(a), w, preferred_element_type=jnp.float32)


def _ln(h, g, b):
    n = h.shape[1]
    jones = jnp.ones((n, 8), jnp.bfloat16)
    mu = jnp.dot(_bf(h), jones, preferred_element_type=jnp.float32)[:, :1] / n
    dd = h - mu
    db = _bf(dd)
    s2 = jnp.dot(db * db, jones, preferred_element_type=jnp.float32)[:, :1]
    return dd * lax.rsqrt(s2 / n + 1e-5) * g + b


def _gelu_new(h):
    c = float(np.sqrt(2.0 / np.pi))
    return 0.5 * h * (1.0 + jnp.tanh(c * (h + 0.044715 * h * h * h)))


def _tc_prep_body(d0, d1, xr, odis, oh):
    deg = d0[0] + d1[0] + 1.0
    dis = lax.rsqrt(deg)
    odis[...] = dis
    x = xr[...]
    oh[0] = x[:, :64] * dis
    oh[1] = x[:, 64:] * dis


def _tc_big_body(a0, a1, xr, disr, w1, b1, g1, e1, wv, bv, wo, bo, g2, e2,
                 wfc, bfc, wp, bp, w2, o_ref):
    dis = disr[...]
    agg = jnp.concatenate([a0[0], a1[0]], axis=1)
    u = dis * agg + (dis * dis) * xr[...]
    h = jax.nn.relu(_dot(u, w1[...]) + b1[...])
    for l in range(2):
        a = _ln(h, g1[l], e1[l])
        v = _dot(a, wv[l]) + bv[l]
        h = h + _dot(v, wo[l]) + bo[l]
        m = _ln(h, g2[l], e2[l])
        f = _gelu_new(_bf(_dot(m, wfc[l]) + bfc[l]))
        h = h + jnp.dot(f, wp[l], preferred_element_type=jnp.float32) + bp[l]
    r = jax.nn.relu(h)
    hwd = _dot(r, w2[...]) * dis
    o_ref[0] = hwd[:, :32]
    o_ref[1] = hwd[:, 32:]


def _tc_fin_body(q0, q1, h0, h1, disr, b2r, o_ref):
    dis = disr[...]
    z0 = dis * (q0[0] + h0[0])
    z1 = dis * (q1[0] + h1[0])
    z = jnp.concatenate([z0, z1], axis=1) + b2r[...]
    col = lax.broadcasted_iota(jnp.int32, z.shape, 1)
    zm = jnp.where(col < 40, z, -1e30)
    mx = jnp.max(zm, axis=1, keepdims=True)
    lse = jnp.log(jnp.sum(jnp.exp(zm - mx), axis=1, keepdims=True)) + mx
    o_ref[...] = (z - lse)[:, :40]


def _row_spec(d):
    return pl.BlockSpec((_BLK, d), lambda i: (i, 0))


def _full(shape):
    nd = len(shape)
    return pl.BlockSpec(shape, lambda i, _nd=nd: (0,) * _nd)


def _part_spec(j, d):
    return pl.BlockSpec((1, _BLK, d), lambda i, _j=j: (_j, i, 0))


def kernel(x, edge_index, W1, b1, ln1_g, ln1_b, Wqkv, bqkv, Wo, bo,
           ln2_g, ln2_b, Wfc, bfc, Wp, bp, W2, b2):
    f32 = jnp.float32
    src, dst = edge_index[0], edge_index[1]
    pad_e = _RPAD * 128 - _E
    padv = jnp.full((pad_e,), _N, jnp.int32)
    src2d = jnp.concatenate([src, padv]).reshape(_RPAD, 128)
    dst2d = jnp.concatenate([dst, padv]).reshape(_RPAD, 128)
    xp = jnp.pad(x, ((0, _NP - _N), (0, 0)))

    degp = _sc_deg(dst2d)
    degp3 = degp[:, :, None]

    dis2, hs0s = pl.pallas_call(
        _tc_prep_body,
        grid=(_G,),
        in_specs=[_part_spec(0, 1), _part_spec(1, 1), _row_spec(128)],
        out_specs=[_row_spec(1), pl.BlockSpec((2, _BLK, 64),
                                              lambda i: (0, i, 0))],
        out_shape=[jax.ShapeDtypeStruct((_NP, 1), f32),
                   jax.ShapeDtypeStruct((2, _NP, 64), f32)],
    )(degp3, degp3, xp)

    ap = _sc_agg(hs0s, src2d, dst2d, 64)

    Wv = _bf(Wqkv[:, :, 2 * _DM:])
    bv = bqkv[:, 2 * _DM:]
    W2p = _bf(jnp.pad(W2, ((0, 0), (0, 24))))
    b2p = jnp.pad(b2, (0, 24))[None]

    hwds = pl.pallas_call(
        _tc_big_body,
        grid=(_G,),
        in_specs=[
            _part_spec(0, 64), _part_spec(1, 64),
            _row_spec(128), _row_spec(1),
            _full((128, _DM)), _full((1, _DM)),
            _full((2, _DM)), _full((2, _DM)),
            _full((2, _DM, _DM)), _full((2, _DM)),
            _full((2, _DM, _DM)), _full((2, _DM)),
            _full((2, _DM)), _full((2, _DM)),
            _full((2, _DM, 3072)), _full((2, 3072)),
            _full((2, 3072, _DM)), _full((2, _DM)),
            _full((_DM, 64)),
        ],
        out_specs=pl.BlockSpec((2, _BLK, 32), lambda i: (0, i, 0)),
        out_shape=jax.ShapeDtypeStruct((2, _NP, 32), f32),
    )(ap, ap, xp, dis2, _bf(W1), b1[None], ln1_g, ln1_b, Wv, bv,
      _bf(Wo), bo, ln2_g, ln2_b, _bf(Wfc), bfc, _bf(Wp), bp, W2p)

    qp = _sc_agg(hwds, src2d, dst2d, 32)

    out = pl.pallas_call(
        _tc_fin_body,
        grid=(_G,),
        in_specs=[
            _part_spec(0, 32), _part_spec(1, 32),
            _part_spec(0, 32), _part_spec(1, 32),
            _row_spec(1), _full((1, 64)),
        ],
        out_specs=_row_spec(40),
        out_shape=jax.ShapeDtypeStruct((_NP, 40), f32),
    )(qp, qp, hwds, hwds, dis2, b2p)

    return out[:_N]


# speedup vs baseline: 1.1094x; 1.0922x over previous
import functools

import jax
import jax.numpy as jnp
import numpy as np
from jax import lax
from jax.experimental import pallas as pl
from jax.experimental.pallas import tpu as pltpu
from jax.experimental.pallas import tpu_sc as plsc

_N = 10000
_E = 320000
_DM = 768
_NP = 10240
_RPAD = 2560
_RPT = _RPAD // 32
_RPT_CS = _RPAD // 16
_ROWS_PT = _NP // 16
_BLK = 1024
_G = _NP // _BLK

_mesh = plsc.VectorSubcoreMesh(core_axis_name="c", subcore_axis_name="s")
_sc_params = pltpu.CompilerParams(use_tc_tiling_on_sc=False)


def _sc_agg(table2, src2d, dst2d, dh, dtype=jnp.float32):
    lanes = 16 if dtype == jnp.float32 else 32

    @functools.partial(
        pl.kernel,
        out_type=jax.ShapeDtypeStruct((2, _NP, dh), dtype),
        mesh=_mesh,
        scratch_types=[
            pltpu.VMEM((_RPT_CS, 128), jnp.int32),
            pltpu.VMEM((_RPT_CS, 128), jnp.int32),
            pltpu.VMEM((4, 128, dh), dtype),
            pltpu.VMEM((128, dh), dtype),
            pltpu.VMEM_SHARED((_NP, dh), dtype),
        ] + [pltpu.SemaphoreType.DMA] * 4,
        compiler_params=_sc_params,
    )
    def k(table_hbm, src_hbm, dst_hbm, out_hbm, sidx, didx, rows4, zbuf, acc,
          *sems):
        c = lax.axis_index("c")
        s = lax.axis_index("s")
        gsems = sems

        @pl.loop(0, 128)
        def _(r):
            for kk in range(dh // lanes):
                zbuf[r, pl.ds(kk * lanes, lanes)] = jnp.zeros((lanes,), dtype)

        @pl.loop(0, 5)
        def _(t):
            pltpu.sync_copy(zbuf, acc.at[pl.ds(s * _ROWS_PT + t * 128, 128)])

        plsc.subcore_barrier()

        pltpu.sync_copy(src_hbm.at[pl.ds(s * _RPT_CS, _RPT_CS)], sidx)
        pltpu.sync_copy(dst_hbm.at[pl.ds(s * _RPT_CS, _RPT_CS)], didx)

        tbl = table_hbm.at[c]
        for b in range(4):
            pltpu.async_copy(tbl.at[sidx.at[b]], rows4.at[b], gsems[b])

        @pl.loop(0, _RPT_CS, step=4)
        def _(j):
            for b in range(4):
                g = j + b
                pltpu.make_async_copy(tbl.at[sidx.at[g]], rows4.at[b],
                                      gsems[b]).wait()
                pltpu.sync_copy(rows4.at[b], acc.at[didx.at[g]], add=True)

                @pl.when(g + 4 < _RPT_CS)
                def _():
                    pltpu.async_copy(tbl.at[sidx.at[g + 4]], rows4.at[b],
                                     gsems[b])

        plsc.subcore_barrier()

        @pl.loop(0, 5)
        def _(t):
            sl = pl.ds(s * _ROWS_PT + t * 128, 128)
            pltpu.sync_copy(acc.at[sl], zbuf)
            pltpu.sync_copy(zbuf, out_hbm.at[c, sl])

    return k(table2, src2d, dst2d)


def _sc_deg(dst2d):
    @functools.partial(
        pl.kernel,
        out_type=jax.ShapeDtypeStruct((2, _NP), jnp.float32),
        mesh=_mesh,
        scratch_types=[
            pltpu.VMEM((_RPT, 128), jnp.int32),
            pltpu.VMEM((128,), jnp.float32),
            pltpu.VMEM((128,), jnp.float32),
            pltpu.VMEM_SHARED((_NP,), jnp.float32),
        ] + [pltpu.SemaphoreType.DMA] * 4,
        compiler_params=_sc_params,
    )
    def k(dst_hbm, out_hbm, didx, ones, buf, acc, *sems):
        c = lax.axis_index("c")
        s = lax.axis_index("s")
        wid = c * 16 + s

        for kk in range(8):
            ones[pl.ds(kk * 16, 16)] = jnp.ones((16,), jnp.float32)
            buf[pl.ds(kk * 16, 16)] = jnp.zeros((16,), jnp.float32)

        @pl.loop(0, 5)
        def _(t):
            pltpu.sync_copy(buf, acc.at[pl.ds(s * _ROWS_PT + t * 128, 128)])

        plsc.subcore_barrier()

        pltpu.sync_copy(dst_hbm.at[pl.ds(wid * _RPT, _RPT)], didx)

        @pl.loop(0, _RPT, step=4)
        def _(j):
            for b in range(4):
                g = j + b

                @pl.when(g >= 4)
                def _():
                    pltpu.make_async_copy(ones, acc.at[didx.at[g - 4]],
                                          sems[b]).wait()
                pltpu.async_copy(ones, acc.at[didx.at[g]], sems[b], add=True)

        for b in range(4):
            pltpu.make_async_copy(ones, acc.at[didx.at[_RPT - 4 + b]],
                                  sems[b]).wait()

        plsc.subcore_barrier()

        @pl.loop(0, 5)
        def _(t):
            sl = pl.ds(s * _ROWS_PT + t * 128, 128)
            pltpu.sync_copy(acc.at[sl], buf)
            pltpu.sync_copy(buf, out_hbm.at[c, sl])

    return k(dst2d)


def _bf(a):
    return a.astype(jnp.bfloat16)


def _dot(a, w):
    return jnp.dot(_bf(a), w, preferred_element_type=jnp.float32)


def _ln(h, g, b):
    n = h.shape[1]
    jones = jnp.ones((n, 8), jnp.bfloat16)
    mu = jnp.dot(_bf(h), jones, preferred_element_type=jnp.float32)[:, :1] / n
    dd = h - mu
    db = _bf(dd)
    s2 = jnp.dot(db * db, jones, preferred_element_type=jnp.float32)[:, :1]
    return dd * lax.rsqrt(s2 / n + 1e-5) * g + b


def _gelu_new(h):
    c = float(np.sqrt(2.0 / np.pi))
    return 0.5 * h * (1.0 + jnp.tanh(c * (h + 0.044715 * h * h * h)))


def _tc_prep_body(d0, d1, xr, odis, oh):
    deg = d0[0] + d1[0] + 1.0
    dis = lax.rsqrt(deg)
    odis[...] = dis
    x = xr[...]
    oh[0] = x[:, :64] * dis
    oh[1] = x[:, 64:] * dis


def _tc_big_body(a0, a1, xr, disr, w1, b1, g1, e1, wv, bv, wo, bo, g2, e2,
                 wfc, bfc, wp, bp, w2, o_ref):
    dis = disr[...]
    agg = jnp.concatenate([a0[0], a1[0]], axis=1)
    u = dis * agg + (dis * dis) * xr[...]
    h = jax.nn.relu(_dot(u, w1[...]) + b1[...])
    for l in range(2):
        a = _ln(h, g1[l], e1[l])
        v = _dot(a, wv[l]) + bv[l]
        h = h + _dot(v, wo[l]) + bo[l]
        m = _ln(h, g2[l], e2[l])
        f = _gelu_new(_bf(_dot(m, wfc[l]) + bfc[l]))
        h = h + jnp.dot(f, wp[l], preferred_element_type=jnp.float32) + bp[l]
    r = jax.nn.relu(h)
    hwd = _dot(r, w2[...]) * dis
    o_ref[0] = _bf(hwd[:, :32])
    o_ref[1] = _bf(hwd[:, 32:])


def _tc_fin_body(q0, q1, h0, h1, disr, b2r, o_ref):
    f32 = jnp.float32
    dis = disr[...]
    z0 = dis * (q0[0].astype(f32) + h0[0].astype(f32))
    z1 = dis * (q1[0].astype(f32) + h1[0].astype(f32))
    z = jnp.concatenate([z0, z1], axis=1) + b2r[...]
    col = lax.broadcasted_iota(jnp.int32, z.shape, 1)
    zm = jnp.where(col < 40, z, -1e30)
    mx = jnp.max(zm, axis=1, keepdims=True)
    lse = jnp.log(jnp.sum(jnp.exp(zm - mx), axis=1, keepdims=True)) + mx
    o_ref[...] = (z - lse)[:, :40]


def _row_spec(d):
    return pl.BlockSpec((_BLK, d), lambda i: (i, 0))


def _full(shape):
    nd = len(shape)
    return pl.BlockSpec(shape, lambda i, _nd=nd: (0,) * _nd)


def _part_spec(j, d):
    return pl.BlockSpec((1, _BLK, d), lambda i, _j=j: (_j, i, 0))


def kernel(x, edge_index, W1, b1, ln1_g, ln1_b, Wqkv, bqkv, Wo, bo,
           ln2_g, ln2_b, Wfc, bfc, Wp, bp, W2, b2):
    f32 = jnp.float32
    src, dst = edge_index[0], edge_index[1]
    pad_e = _RPAD * 128 - _E
    padv = jnp.full((pad_e,), _N, jnp.int32)
    src2d = jnp.concatenate([src, padv]).reshape(_RPAD, 128)
    dst2d = jnp.concatenate([dst, padv]).reshape(_RPAD, 128)
    xp = jnp.pad(x, ((0, _NP - _N), (0, 0)))

    degp = _sc_deg(dst2d)
    degp3 = degp[:, :, None]

    dis2, hs0s = pl.pallas_call(
        _tc_prep_body,
        grid=(_G,),
        in_specs=[_part_spec(0, 1), _part_spec(1, 1), _row_spec(128)],
        out_specs=[_row_spec(1), pl.BlockSpec((2, _BLK, 64),
                                              lambda i: (0, i, 0))],
        out_shape=[jax.ShapeDtypeStruct((_NP, 1), f32),
                   jax.ShapeDtypeStruct((2, _NP, 64), f32)],
    )(degp3, degp3, xp)

    ap = _sc_agg(hs0s, src2d, dst2d, 64)

    Wv = _bf(Wqkv[:, :, 2 * _DM:])
    bv = bqkv[:, 2 * _DM:]
    W2p = _bf(jnp.pad(W2, ((0, 0), (0, 24))))
    b2p = jnp.pad(b2, (0, 24))[None]

    hwds = pl.pallas_call(
        _tc_big_body,
        grid=(_G,),
        in_specs=[
            _part_spec(0, 64), _part_spec(1, 64),
            _row_spec(128), _row_spec(1),
            _full((128, _DM)), _full((1, _DM)),
            _full((2, _DM)), _full((2, _DM)),
            _full((2, _DM, _DM)), _full((2, _DM)),
            _full((2, _DM, _DM)), _full((2, _DM)),
            _full((2, _DM)), _full((2, _DM)),
            _full((2, _DM, 3072)), _full((2, 3072)),
            _full((2, 3072, _DM)), _full((2, _DM)),
            _full((_DM, 64)),
        ],
        out_specs=pl.BlockSpec((2, _BLK, 32), lambda i: (0, i, 0)),
        out_shape=jax.ShapeDtypeStruct((2, _NP, 32), jnp.bfloat16),
    )(ap, ap, xp, dis2, _bf(W1), b1[None], ln1_g, ln1_b, Wv, bv,
      _bf(Wo), bo, ln2_g, ln2_b, _bf(Wfc), bfc, _bf(Wp), bp, W2p)

    qp = _sc_agg(hwds, src2d, dst2d, 32, jnp.bfloat16)

    out = pl.pallas_call(
        _tc_fin_body,
        grid=(_G,),
        in_specs=[
            _part_spec(0, 32), _part_spec(1, 32),
            _part_spec(0, 32), _part_spec(1, 32),
            _row_spec(1), _full((1, 64)),
        ],
        out_specs=_row_spec(40),
        out_shape=jax.ShapeDtypeStruct((_NP, 40), f32),
    )(qp, qp, hwds, hwds, dis2, b2p)

    return out[:_N]


# speedup vs baseline: 1.3111x; 1.1818x over previous
import functools

import jax
import jax.numpy as jnp
import numpy as np
from jax import lax
from jax.experimental import pallas as pl
from jax.experimental.pallas import tpu as pltpu
from jax.experimental.pallas import tpu_sc as plsc

_N = 10000
_E = 320000
_DM = 768
_NP = 10240
_RPAD = 2560
_RPT = _RPAD // 32
_RPT_CS = _RPAD // 16
_ROWS_PT = _NP // 16
_BLK = 1024
_G = _NP // _BLK

_mesh = plsc.VectorSubcoreMesh(core_axis_name="c", subcore_axis_name="s")
_sc_params = pltpu.CompilerParams(use_tc_tiling_on_sc=False)


def _sc_agg(table2, src2d, dst2d, dh, dtype=jnp.float32):
    lanes = 16 if dtype == jnp.float32 else 32

    @functools.partial(
        pl.kernel,
        out_type=jax.ShapeDtypeStruct((2, _NP, dh), dtype),
        mesh=_mesh,
        scratch_types=[
            pltpu.VMEM((_RPT_CS, 128), jnp.int32),
            pltpu.VMEM((_RPT_CS, 128), jnp.int32),
            pltpu.VMEM((4, 128, dh), dtype),
            pltpu.VMEM((128, dh), dtype),
            pltpu.VMEM_SHARED((_NP, dh), dtype),
        ] + [pltpu.SemaphoreType.DMA] * 4,
        compiler_params=_sc_params,
    )
    def k(table_hbm, src_hbm, dst_hbm, out_hbm, sidx, didx, rows4, zbuf, acc,
          *sems):
        c = lax.axis_index("c")
        s = lax.axis_index("s")
        gsems = sems

        @pl.loop(0, 128)
        def _(r):
            for kk in range(dh // lanes):
                zbuf[r, pl.ds(kk * lanes, lanes)] = jnp.zeros((lanes,), dtype)

        @pl.loop(0, 5)
        def _(t):
            pltpu.sync_copy(zbuf, acc.at[pl.ds(s * _ROWS_PT + t * 128, 128)])

        plsc.subcore_barrier()

        pltpu.sync_copy(src_hbm.at[pl.ds(s * _RPT_CS, _RPT_CS)], sidx)
        pltpu.sync_copy(dst_hbm.at[pl.ds(s * _RPT_CS, _RPT_CS)], didx)

        tbl = table_hbm.at[c]
        for b in range(4):
            pltpu.async_copy(tbl.at[sidx.at[b]], rows4.at[b], gsems[b])

        @pl.loop(0, _RPT_CS, step=4)
        def _(j):
            for b in range(4):
                g = j + b
                pltpu.make_async_copy(tbl.at[sidx.at[g]], rows4.at[b],
                                      gsems[b]).wait()
                pltpu.sync_copy(rows4.at[b], acc.at[didx.at[g]], add=True)

                @pl.when(g + 4 < _RPT_CS)
                def _():
                    pltpu.async_copy(tbl.at[sidx.at[g + 4]], rows4.at[b],
                                     gsems[b])

        plsc.subcore_barrier()

        @pl.loop(0, 5)
        def _(t):
            sl = pl.ds(s * _ROWS_PT + t * 128, 128)
            pltpu.sync_copy(acc.at[sl], zbuf)
            pltpu.sync_copy(zbuf, out_hbm.at[c, sl])

    return k(table2, src2d, dst2d)


def _sc_deg(dst2d):
    @functools.partial(
        pl.kernel,
        out_type=jax.ShapeDtypeStruct((2, _NP), jnp.float32),
        mesh=_mesh,
        scratch_types=[
            pltpu.VMEM((_RPT, 128), jnp.int32),
            pltpu.VMEM((128,), jnp.float32),
            pltpu.VMEM((128,), jnp.float32),
            pltpu.VMEM_SHARED((_NP,), jnp.float32),
        ] + [pltpu.SemaphoreType.DMA] * 4,
        compiler_params=_sc_params,
    )
    def k(dst_hbm, out_hbm, didx, ones, buf, acc, *sems):
        c = lax.axis_index("c")
        s = lax.axis_index("s")
        wid = c * 16 + s

        for kk in range(8):
            ones[pl.ds(kk * 16, 16)] = jnp.ones((16,), jnp.float32)
            buf[pl.ds(kk * 16, 16)] = jnp.zeros((16,), jnp.float32)

        @pl.loop(0, 5)
        def _(t):
            pltpu.sync_copy(buf, acc.at[pl.ds(s * _ROWS_PT + t * 128, 128)])

        plsc.subcore_barrier()

        pltpu.sync_copy(dst_hbm.at[pl.ds(wid * _RPT, _RPT)], didx)

        @pl.loop(0, _RPT, step=4)
        def _(j):
            for b in range(4):
                g = j + b

                @pl.when(g >= 4)
                def _():
                    pltpu.make_async_copy(ones, acc.at[didx.at[g - 4]],
                                          sems[b]).wait()
                pltpu.async_copy(ones, acc.at[didx.at[g]], sems[b], add=True)

        for b in range(4):
            pltpu.make_async_copy(ones, acc.at[didx.at[_RPT - 4 + b]],
                                  sems[b]).wait()

        plsc.subcore_barrier()

        @pl.loop(0, 5)
        def _(t):
            sl = pl.ds(s * _ROWS_PT + t * 128, 128)
            pltpu.sync_copy(acc.at[sl], buf)
            pltpu.sync_copy(buf, out_hbm.at[c, sl])

    return k(dst2d)


def _bf(a):
    return a.astype(jnp.bfloat16)


def _dot(a, w):
    return jnp.dot(_bf(a), w, preferred_element_type=jnp.float32)


def _ln(h, g, b):
    n = h.shape[1]
    jones = jnp.ones((n, 8), jnp.bfloat16)
    mu = jnp.dot(_bf(h), jones, preferred_element_type=jnp.float32)[:, :1] / n
    dd = h - mu
    db = _bf(dd)
    s2 = jnp.dot(db * db, jones, preferred_element_type=jnp.float32)[:, :1]
    return dd * lax.rsqrt(s2 / n + 1e-5) * g + b


def _gelu_new(h):
    c = float(np.sqrt(2.0 / np.pi))
    return 0.5 * h * (1.0 + jnp.tanh(c * (h + 0.044715 * h * h * h)))


def _tc_prep_body(d0, d1, xr, odis, oh):
    deg = d0[0] + d1[0] + 1.0
    dis = lax.rsqrt(deg)
    odis[...] = dis
    x = xr[...]
    oh[0] = _bf(x[:, :64] * dis)
    oh[1] = _bf(x[:, 64:] * dis)


def _tc_big_body(a0, a1, xr, disr, w1, b1, g1, e1, wv, bv, wo, bo, g2, e2,
                 wfc, bfc, wp, bp, w2, o_ref):
    dis = disr[...]
    agg = jnp.concatenate([a0[0], a1[0]], axis=1).astype(jnp.float32)
    u = dis * agg + (dis * dis) * xr[...]
    h = jax.nn.relu(_dot(u, w1[...]) + b1[...])
    for l in range(2):
        a = _ln(h, g1[l], e1[l])
        v = _dot(a, wv[l]) + bv[l]
        h = h + _dot(v, wo[l]) + bo[l]
        m = _ln(h, g2[l], e2[l])
        f = _gelu_new(_bf(_dot(m, wfc[l]) + bfc[l]))
        h = h + jnp.dot(f, wp[l], preferred_element_type=jnp.float32) + bp[l]
    r = jax.nn.relu(h)
    hwd = _dot(r, w2[...]) * dis
    o_ref[0] = _bf(hwd[:, :32])
    o_ref[1] = _bf(hwd[:, 32:])


def _tc_fin_body(q0, q1, h0, h1, disr, b2r, o_ref):
    f32 = jnp.float32
    dis = disr[...]
    z0 = dis * (q0[0].astype(f32) + h0[0].astype(f32))
    z1 = dis * (q1[0].astype(f32) + h1[0].astype(f32))
    z = jnp.concatenate([z0, z1], axis=1) + b2r[...]
    col = lax.broadcasted_iota(jnp.int32, z.shape, 1)
    zm = jnp.where(col < 40, z, -1e30)
    mx = jnp.max(zm, axis=1, keepdims=True)
    lse = jnp.log(jnp.sum(jnp.exp(zm - mx), axis=1, keepdims=True)) + mx
    o_ref[...] = (z - lse)[:, :40]


def _row_spec(d):
    return pl.BlockSpec((_BLK, d), lambda i: (i, 0))


def _full(shape):
    nd = len(shape)
    return pl.BlockSpec(shape, lambda i, _nd=nd: (0,) * _nd)


def _part_spec(j, d):
    return pl.BlockSpec((1, _BLK, d), lambda i, _j=j: (_j, i, 0))


def kernel(x, edge_index, W1, b1, ln1_g, ln1_b, Wqkv, bqkv, Wo, bo,
           ln2_g, ln2_b, Wfc, bfc, Wp, bp, W2, b2):
    f32 = jnp.float32
    src, dst = edge_index[0], edge_index[1]
    pad_e = _RPAD * 128 - _E
    padv = jnp.full((pad_e,), _N, jnp.int32)
    src2d = jnp.concatenate([src, padv]).reshape(_RPAD, 128)
    dst2d = jnp.concatenate([dst, padv]).reshape(_RPAD, 128)
    xp = jnp.pad(x, ((0, _NP - _N), (0, 0)))

    degp = _sc_deg(dst2d)
    degp3 = degp[:, :, None]

    dis2, hs0s = pl.pallas_call(
        _tc_prep_body,
        grid=(_G,),
        in_specs=[_part_spec(0, 1), _part_spec(1, 1), _row_spec(128)],
        out_specs=[_row_spec(1), pl.BlockSpec((2, _BLK, 64),
                                              lambda i: (0, i, 0))],
        out_shape=[jax.ShapeDtypeStruct((_NP, 1), f32),
                   jax.ShapeDtypeStruct((2, _NP, 64), jnp.bfloat16)],
    )(degp3, degp3, xp)

    ap = _sc_agg(hs0s, src2d, dst2d, 64, jnp.bfloat16)

    Wv = _bf(Wqkv[:, :, 2 * _DM:])
    bv = bqkv[:, 2 * _DM:]
    W2p = _bf(jnp.pad(W2, ((0, 0), (0, 24))))
    b2p = jnp.pad(b2, (0, 24))[None]

    hwds = pl.pallas_call(
        _tc_big_body,
        grid=(_G,),
        in_specs=[
            _part_spec(0, 64), _part_spec(1, 64),
            _row_spec(128), _row_spec(1),
            _full((128, _DM)), _full((1, _DM)),
            _full((2, _DM)), _full((2, _DM)),
            _full((2, _DM, _DM)), _full((2, _DM)),
            _full((2, _DM, _DM)), _full((2, _DM)),
            _full((2, _DM)), _full((2, _DM)),
            _full((2, _DM, 3072)), _full((2, 3072)),
            _full((2, 3072, _DM)), _full((2, _DM)),
            _full((_DM, 64)),
        ],
        out_specs=pl.BlockSpec((2, _BLK, 32), lambda i: (0, i, 0)),
        out_shape=jax.ShapeDtypeStruct((2, _NP, 32), jnp.bfloat16),
    )(ap, ap, xp, dis2, _bf(W1), b1[None], ln1_g, ln1_b, Wv, bv,
      _bf(Wo), bo, ln2_g, ln2_b, _bf(Wfc), bfc, _bf(Wp), bp, W2p)

    qp = _sc_agg(hwds, src2d, dst2d, 32, jnp.bfloat16)

    out = pl.pallas_call(
        _tc_fin_body,
        grid=(_G,),
        in_specs=[
            _part_spec(0, 32), _part_spec(1, 32),
            _part_spec(0, 32), _part_spec(1, 32),
            _row_spec(1), _full((1, 64)),
        ],
        out_specs=_row_spec(40),
        out_shape=jax.ShapeDtypeStruct((_NP, 40), f32),
    )(qp, qp, hwds, hwds, dis2, b2p)

    return out[:_N]
